# baseline (speedup 1.0000x reference)
import jax
import jax.numpy as jnp
from jax import lax
from jax.experimental import pallas as pl
from jax.experimental.pallas import tpu as pltpu
from jax.experimental.pallas import tpu_sc as plsc

N_KNOTS = 2048
N_BIN = N_KNOTS - 1
CHANNELS = 512
N_QUERY = 32768

NC = 2
NS = 16
NW = NC * NS
L = 16

QPW = N_QUERY // NW
CQ = L
NCHUNK = QPW // CQ
NJ2 = CHANNELS // (2 * L)


def _spline_body(t_hbm, knots_hbm, ft_hbm, out_hbm,
                 t_v, idx_v, frac_v, knots_v,
                 rt, out_v, gsem0, gsem1, osem0, osem1):
    wid = lax.axis_index("s") * NC + lax.axis_index("c")
    base = wid * QPW
    gsems = (gsem0, gsem1)
    osems = (osem0, osem1)

    pltpu.sync_copy(t_hbm.at[pl.ds(base, QPW)], t_v)
    pltpu.sync_copy(knots_hbm, knots_v)

    k0 = jnp.full((L,), knots_v[pl.ds(0, L)][0])
    rng = jnp.full((L,), knots_v[pl.ds(N_KNOTS - L, L)][L - 1]) - k0
    nbin = jnp.full((L,), jnp.float32(N_BIN))
    factor = nbin / rng
    step = rng / nbin

    @plsc.parallel_loop(0, QPW // L, 1, unroll=4)
    def idx_step(i):
        tv = t_v[pl.ds(i * L, L)]
        iv = (factor * tv).astype(jnp.int32)
        idx_v[pl.ds(i * L, L)] = iv
        frac_v[pl.ds(i * L, L)] = tv - (k0 + iv.astype(jnp.float32) * step)

    def fire_gather(g, b):
        iv = idx_v[pl.ds(g * CQ, CQ)]
        pltpu.async_copy(ft_hbm.at[iv], rt.at[b], gsems[b])

    def wait_gather(g, b):
        iv = idx_v[pl.ds(g * CQ, CQ)]
        pltpu.make_async_copy(ft_hbm.at[iv], rt.at[b], gsems[b]).wait()

    def out_slice(g):
        return out_hbm.at[pl.ds(base + g * CQ, CQ)]

    def compute(g, b):
        fv = frac_v[pl.ds(g * CQ, CQ)]
        fs = [jnp.full((L,), fv[q]) for q in range(CQ)]
        fsb = []
        for q in range(CQ):
            h = (plsc.bitcast(fs[q], jnp.uint32) + jnp.uint32(0x8000)) >> jnp.uint32(16)
            fsb.append(plsc.bitcast((h << jnp.uint32(16)) | h, jnp.bfloat16))

        @plsc.parallel_loop(0, NJ2, 1, unroll=2)
        def j_step(j):
            s0 = pl.ds(j * 2 * L, L)
            s1 = pl.ds(j * 2 * L + L, L)
            for q in range(CQ):
                fb = fsb[q]
                a0 = plsc.bitcast(rt[b, q, pl.ds(j * 2 * L, L)], jnp.float32)
                a1 = plsc.bitcast(rt[b, q, pl.ds(j * 2 * L + L, L)], jnp.float32)
                bv = plsc.bitcast(rt[b, q, pl.ds(512 + j * L, L)], jnp.bfloat16)
                cv = plsc.bitcast(rt[b, q, pl.ds(768 + j * L, L)], jnp.bfloat16)
                dv = plsc.bitcast(rt[b, q, pl.ds(1024 + j * L, L)], jnp.bfloat16)
                inner = bv + fb * (cv + fb * dv)
                u = plsc.bitcast(inner, jnp.uint32)
                i0 = plsc.bitcast(u << jnp.uint32(16), jnp.float32)
                i1 = plsc.bitcast(u & jnp.uint32(0xFFFF0000), jnp.float32)
                f = fs[q]
                out_v[b, q, s0] = a0 + f * i0
                out_v[b, q, s1] = a1 + f * i1

    fire_gather(0, 0)
    fire_gather(1, 1)

    def pair_step(gg, _):
        for b in range(2):
            g = 2 * gg + b
            wait_gather(g, b)

            @pl.when(gg > 0)
            def _():
                pltpu.make_async_copy(out_v.at[b], out_slice(g - 2), osems[b]).wait()

            compute(g, b)

            @pl.when(g + 2 < NCHUNK)
            def _():
                fire_gather(g + 2, b)

            pltpu.async_copy(out_v.at[b], out_slice(g), osems[b])
        return 0

    lax.fori_loop(0, NCHUNK // 2, pair_step, 0)

    pltpu.make_async_copy(out_v.at[0], out_slice(NCHUNK - 2), osems[0]).wait()
    pltpu.make_async_copy(out_v.at[1], out_slice(NCHUNK - 1), osems[1]).wait()


def _shuffle_bf16(x):
    n = x.shape[0]
    u = lax.bitcast_convert_type(x, jnp.uint32) + jnp.uint32(0x8000)
    u = u.reshape(n, CHANNELS // 32, 2, L)
    return ((u[:, :, 0, :] >> jnp.uint32(16))
            | (u[:, :, 1, :] & jnp.uint32(0xFFFF0000))).reshape(n, CHANNELS // 2)


@jax.jit
def kernel(t, knots, a, b, c, d):
    mesh = plsc.VectorSubcoreMesh(core_axis_name="c", subcore_axis_name="s")
    run = pl.kernel(
        _spline_body,
        out_type=jax.ShapeDtypeStruct((N_QUERY, CHANNELS), jnp.float32),
        mesh=mesh,
        compiler_params=pltpu.CompilerParams(needs_layout_passes=False),
        scratch_types=[
            pltpu.VMEM((QPW,), jnp.float32),
            pltpu.VMEM((QPW,), jnp.int32),
            pltpu.VMEM((QPW,), jnp.float32),
            pltpu.VMEM((N_KNOTS,), jnp.float32),
            pltpu.VMEM((2, CQ, 1280), jnp.uint32),
            pltpu.VMEM((2, CQ, CHANNELS), jnp.float32),
            pltpu.SemaphoreType.DMA,
            pltpu.SemaphoreType.DMA,
            pltpu.SemaphoreType.DMA,
            pltpu.SemaphoreType.DMA,
        ],
    )
    ft = jnp.concatenate(
        [lax.bitcast_convert_type(a, jnp.uint32),
         _shuffle_bf16(b), _shuffle_bf16(c), _shuffle_bf16(d)], axis=1)
    return run(t, knots, ft)


# speedup vs baseline: 1.1375x; 1.1375x over previous
import jax
import jax.numpy as jnp
from jax import lax
from jax.experimental import pallas as pl
from jax.experimental.pallas import tpu as pltpu
from jax.experimental.pallas import tpu_sc as plsc

N_KNOTS = 2048
N_BIN = N_KNOTS - 1
CHANNELS = 512
N_QUERY = 32768

NC = 2
NS = 16
NW = NC * NS
L = 16

QPW = N_QUERY // NW
CQ = L
NCHUNK = QPW // CQ
NJ2 = CHANNELS // (2 * L)


def _spline_body(t_hbm, knots_hbm, ft_hbm, out_hbm,
                 t_v, idx_v, frac_v, knots_v,
                 rt, out_v, gsem0, gsem1, osem0, osem1):
    wid = lax.axis_index("s") * NC + lax.axis_index("c")
    base = wid * QPW
    gsems = (gsem0, gsem1)
    osems = (osem0, osem1)

    pltpu.sync_copy(t_hbm.at[pl.ds(base, QPW)], t_v)
    pltpu.sync_copy(knots_hbm, knots_v)

    k0 = jnp.full((L,), knots_v[pl.ds(0, L)][0])
    rng = jnp.full((L,), knots_v[pl.ds(N_KNOTS - L, L)][L - 1]) - k0
    nbin = jnp.full((L,), jnp.float32(N_BIN))
    factor = nbin / rng
    step = rng / nbin

    @plsc.parallel_loop(0, QPW // L, 1, unroll=4)
    def idx_step(i):
        tv = t_v[pl.ds(i * L, L)]
        iv = (factor * tv).astype(jnp.int32)
        idx_v[pl.ds(i * L, L)] = iv
        frac_v[pl.ds(i * L, L)] = tv - (k0 + iv.astype(jnp.float32) * step)

    def fire_gather(g, b):
        iv = idx_v[pl.ds(g * CQ, CQ)]
        pltpu.async_copy(ft_hbm.at[iv], rt.at[b], gsems[b])

    def wait_gather(g, b):
        iv = idx_v[pl.ds(g * CQ, CQ)]
        pltpu.make_async_copy(ft_hbm.at[iv], rt.at[b], gsems[b]).wait()

    def out_slice(g):
        return out_hbm.at[pl.ds(base + g * CQ, CQ)]

    def compute(g, b):
        fv = frac_v[pl.ds(g * CQ, CQ)]
        fs = [jnp.full((L,), fv[q]) for q in range(CQ)]
        fsb = []
        for q in range(CQ):
            h = (plsc.bitcast(fs[q], jnp.uint32) + jnp.uint32(0x8000)) >> jnp.uint32(16)
            fsb.append(plsc.bitcast((h << jnp.uint32(16)) | h, jnp.bfloat16))

        @plsc.parallel_loop(0, NJ2, 1, unroll=2)
        def j_step(j):
            s0 = pl.ds(j * 2 * L, L)
            s1 = pl.ds(j * 2 * L + L, L)
            for q in range(CQ):
                fb = fsb[q]
                a0 = plsc.bitcast(rt[b, q, pl.ds(j * 2 * L, L)], jnp.float32)
                a1 = plsc.bitcast(rt[b, q, pl.ds(j * 2 * L + L, L)], jnp.float32)
                bv = plsc.bitcast(rt[b, q, pl.ds(512 + j * L, L)], jnp.bfloat16)
                cv = plsc.bitcast(rt[b, q, pl.ds(768 + j * L, L)], jnp.bfloat16)
                dv = plsc.bitcast(rt[b, q, pl.ds(1024 + j * L, L)], jnp.bfloat16)
                inner = bv + fb * (cv + fb * dv)
                u = plsc.bitcast(inner, jnp.uint32)
                i0 = plsc.bitcast(u << jnp.uint32(16), jnp.float32)
                i1 = plsc.bitcast(u & jnp.uint32(0xFFFF0000), jnp.float32)
                f = fs[q]
                out_v[b, q, s0] = a0 + f * i0
                out_v[b, q, s1] = a1 + f * i1

    fire_gather(0, 0)
    fire_gather(1, 1)

    def pair_step(gg, _):
        for b in range(2):
            g = 2 * gg + b
            wait_gather(g, b)

            @pl.when(gg > 0)
            def _():
                pltpu.make_async_copy(out_v.at[b], out_slice(g - 2), osems[b]).wait()

            compute(g, b)

            @pl.when(g + 2 < NCHUNK)
            def _():
                fire_gather(g + 2, b)

            pltpu.async_copy(out_v.at[b], out_slice(g), osems[b])
        return 0

    lax.fori_loop(0, NCHUNK // 2, pair_step, 0)

    pltpu.make_async_copy(out_v.at[0], out_slice(NCHUNK - 2), osems[0]).wait()
    pltpu.make_async_copy(out_v.at[1], out_slice(NCHUNK - 1), osems[1]).wait()


def _shuffle_bf16(x):
    n = x.shape[0]
    x = x.astype(jnp.bfloat16).reshape(n, CHANNELS // 32, 2, L)
    x = x.transpose(0, 1, 3, 2).reshape(n, CHANNELS // 2, 2)
    return lax.bitcast_convert_type(x, jnp.uint32)


@jax.jit
def kernel(t, knots, a, b, c, d):
    mesh = plsc.VectorSubcoreMesh(core_axis_name="c", subcore_axis_name="s")
    run = pl.kernel(
        _spline_body,
        out_type=jax.ShapeDtypeStruct((N_QUERY, CHANNELS), jnp.float32),
        mesh=mesh,
        compiler_params=pltpu.CompilerParams(needs_layout_passes=False),
        scratch_types=[
            pltpu.VMEM((QPW,), jnp.float32),
            pltpu.VMEM((QPW,), jnp.int32),
            pltpu.VMEM((QPW,), jnp.float32),
            pltpu.VMEM((N_KNOTS,), jnp.float32),
            pltpu.VMEM((2, CQ, 1280), jnp.uint32),
            pltpu.VMEM((2, CQ, CHANNELS), jnp.float32),
            pltpu.SemaphoreType.DMA,
            pltpu.SemaphoreType.DMA,
            pltpu.SemaphoreType.DMA,
            pltpu.SemaphoreType.DMA,
        ],
    )
    ft = jnp.concatenate(
        [lax.bitcast_convert_type(a, jnp.uint32),
         _shuffle_bf16(b), _shuffle_bf16(c), _shuffle_bf16(d)], axis=1)
    return run(t, knots, ft)


# speedup vs baseline: 1.2762x; 1.1219x over previous
import jax
import jax.numpy as jnp
from jax import lax
from jax.experimental import pallas as pl
from jax.experimental.pallas import tpu as pltpu
from jax.experimental.pallas import tpu_sc as plsc

N_KNOTS = 2048
N_BIN = N_KNOTS - 1
CHANNELS = 512
N_QUERY = 32768

NC = 2
NS = 16
NW = NC * NS
L = 16

QPW = N_QUERY // NW
CQ = 2 * L
NCHUNK = QPW // CQ
NJ2 = CHANNELS // (2 * L)


def _spline_body(t_hbm, knots_hbm, ft_hbm, out_hbm,
                 t_v, idx_v, frac_v, knots_v,
                 rt, out_v, gsem0, gsem1, osem0, osem1):
    wid = lax.axis_index("s") * NC + lax.axis_index("c")
    base = wid * QPW
    gsems = (gsem0, gsem1)
    osems = (osem0, osem1)

    pltpu.sync_copy(t_hbm.at[pl.ds(base, QPW)], t_v)
    pltpu.sync_copy(knots_hbm, knots_v)

    k0 = jnp.full((L,), knots_v[pl.ds(0, L)][0])
    rng = jnp.full((L,), knots_v[pl.ds(N_KNOTS - L, L)][L - 1]) - k0
    nbin = jnp.full((L,), jnp.float32(N_BIN))
    factor = nbin / rng
    step = rng / nbin

    @plsc.parallel_loop(0, QPW // L, 1, unroll=4)
    def idx_step(i):
        tv = t_v[pl.ds(i * L, L)]
        iv = (factor * tv).astype(jnp.int32)
        idx_v[pl.ds(i * L, L)] = iv
        frac_v[pl.ds(i * L, L)] = tv - (k0 + iv.astype(jnp.float32) * step)

    def fire_gather(g, b):
        pltpu.async_copy(ft_hbm.at[idx_v.at[pl.ds(g * CQ, CQ)]], rt.at[b], gsems[b])

    def wait_gather(g, b):
        pltpu.make_async_copy(ft_hbm.at[idx_v.at[pl.ds(g * CQ, CQ)]], rt.at[b], gsems[b]).wait()

    def out_slice(g):
        return out_hbm.at[pl.ds(base + g * CQ, CQ)]

    def compute(g, b):
        fv0 = frac_v[pl.ds(g * CQ, L)]
        fv1 = frac_v[pl.ds(g * CQ + L, L)]
        fs = ([jnp.full((L,), fv0[q]) for q in range(L)]
              + [jnp.full((L,), fv1[q]) for q in range(L)])
        fsb = []
        for q in range(CQ):
            h = (plsc.bitcast(fs[q], jnp.uint32) + jnp.uint32(0x8000)) >> jnp.uint32(16)
            fsb.append(plsc.bitcast((h << jnp.uint32(16)) | h, jnp.bfloat16))

        @plsc.parallel_loop(0, NJ2, 1, unroll=2)
        def j_step(j):
            s0 = pl.ds(j * 2 * L, L)
            s1 = pl.ds(j * 2 * L + L, L)
            for q in range(CQ):
                fb = fsb[q]
                a0 = plsc.bitcast(rt[b, q, pl.ds(j * 2 * L, L)], jnp.float32)
                a1 = plsc.bitcast(rt[b, q, pl.ds(j * 2 * L + L, L)], jnp.float32)
                bv = plsc.bitcast(rt[b, q, pl.ds(512 + j * L, L)], jnp.bfloat16)
                cv = plsc.bitcast(rt[b, q, pl.ds(768 + j * L, L)], jnp.bfloat16)
                dv = plsc.bitcast(rt[b, q, pl.ds(1024 + j * L, L)], jnp.bfloat16)
                inner = bv + fb * (cv + fb * dv)
                u = plsc.bitcast(inner, jnp.uint32)
                i0 = plsc.bitcast(u << jnp.uint32(16), jnp.float32)
                i1 = plsc.bitcast(u & jnp.uint32(0xFFFF0000), jnp.float32)
                f = fs[q]
                out_v[b, q, s0] = a0 + f * i0
                out_v[b, q, s1] = a1 + f * i1

    fire_gather(0, 0)
    fire_gather(1, 1)

    def pair_step(gg, _):
        for b in range(2):
            g = 2 * gg + b
            wait_gather(g, b)

            @pl.when(gg > 0)
            def _():
                pltpu.make_async_copy(out_v.at[b], out_slice(g - 2), osems[b]).wait()

            compute(g, b)

            @pl.when(g + 2 < NCHUNK)
            def _():
                fire_gather(g + 2, b)

            pltpu.async_copy(out_v.at[b], out_slice(g), osems[b])
        return 0

    lax.fori_loop(0, NCHUNK // 2, pair_step, 0)

    pltpu.make_async_copy(out_v.at[0], out_slice(NCHUNK - 2), osems[0]).wait()
    pltpu.make_async_copy(out_v.at[1], out_slice(NCHUNK - 1), osems[1]).wait()


def _shuffle_bf16(x):
    n = x.shape[0]
    x = x.astype(jnp.bfloat16).reshape(n, CHANNELS // 32, 2, L)
    x = x.transpose(0, 1, 3, 2).reshape(n, CHANNELS // 2, 2)
    return lax.bitcast_convert_type(x, jnp.uint32)


@jax.jit
def kernel(t, knots, a, b, c, d):
    mesh = plsc.VectorSubcoreMesh(core_axis_name="c", subcore_axis_name="s")
    run = pl.kernel(
        _spline_body,
        out_type=jax.ShapeDtypeStruct((N_QUERY, CHANNELS), jnp.float32),
        mesh=mesh,
        compiler_params=pltpu.CompilerParams(needs_layout_passes=False),
        scratch_types=[
            pltpu.VMEM((QPW,), jnp.float32),
            pltpu.VMEM((QPW,), jnp.int32),
            pltpu.VMEM((QPW,), jnp.float32),
            pltpu.VMEM((N_KNOTS,), jnp.float32),
            pltpu.VMEM((2, CQ, 1280), jnp.uint32),
            pltpu.VMEM((2, CQ, CHANNELS), jnp.float32),
            pltpu.SemaphoreType.DMA,
            pltpu.SemaphoreType.DMA,
            pltpu.SemaphoreType.DMA,
            pltpu.SemaphoreType.DMA,
        ],
    )
    ft = jnp.concatenate(
        [lax.bitcast_convert_type(a, jnp.uint32),
         _shuffle_bf16(b), _shuffle_bf16(c), _shuffle_bf16(d)], axis=1)
    return run(t, knots, ft)


# speedup vs baseline: 1.4218x; 1.1141x over previous
import jax
import jax.numpy as jnp
from jax import lax
from jax.experimental import pallas as pl
from jax.experimental.pallas import tpu as pltpu
from jax.experimental.pallas import tpu_sc as plsc

N_KNOTS = 2048
N_BIN = N_KNOTS - 1
CHANNELS = 512
N_QUERY = 32768

NC = 2
NS = 16
NW = NC * NS
L = 16

QPW = N_QUERY // NW
CQ = 2 * L
NCHUNK = QPW // CQ
NJ2 = CHANNELS // (2 * L)


def _spline_body(t_hbm, knots_hbm, ft_hbm, out_hbm,
                 t_v, idx_v, frac_v, knots_v,
                 rt, out_v, gsem0, gsem1, osem0, osem1):
    wid = lax.axis_index("s") * NC + lax.axis_index("c")
    base = wid * QPW
    gsems = (gsem0, gsem1)
    osems = (osem0, osem1)

    pltpu.sync_copy(t_hbm.at[pl.ds(base, QPW)], t_v)
    pltpu.sync_copy(knots_hbm, knots_v)

    k0 = jnp.full((L,), knots_v[pl.ds(0, L)][0])
    rng = jnp.full((L,), knots_v[pl.ds(N_KNOTS - L, L)][L - 1]) - k0
    nbin = jnp.full((L,), jnp.float32(N_BIN))
    factor = nbin / rng
    step = rng / nbin

    @plsc.parallel_loop(0, QPW // L, 1, unroll=4)
    def idx_step(i):
        tv = t_v[pl.ds(i * L, L)]
        iv = (factor * tv).astype(jnp.int32)
        idx_v[pl.ds(i * L, L)] = iv
        frac_v[pl.ds(i * L, L)] = tv - (k0 + iv.astype(jnp.float32) * step)

    def fire_gather(g, b):
        pltpu.async_copy(ft_hbm.at[idx_v.at[pl.ds(g * CQ, CQ)]], rt.at[b], gsems[b])

    def wait_gather(g, b):
        pltpu.make_async_copy(ft_hbm.at[idx_v.at[pl.ds(g * CQ, CQ)]], rt.at[b], gsems[b]).wait()

    def out_slice(g):
        return out_hbm.at[pl.ds(base + g * CQ, CQ)]

    def compute(g, b):
        fv0 = frac_v[pl.ds(g * CQ, L)]
        fv1 = frac_v[pl.ds(g * CQ + L, L)]
        fs = ([jnp.full((L,), fv0[q]) for q in range(L)]
              + [jnp.full((L,), fv1[q]) for q in range(L)])
        fsb = []
        for q in range(CQ):
            h = (plsc.bitcast(fs[q], jnp.uint32) + jnp.uint32(0x8000)) >> jnp.uint32(16)
            fsb.append(plsc.bitcast((h << jnp.uint32(16)) | h, jnp.bfloat16))

        H = CHANNELS // 2

        @plsc.parallel_loop(0, NJ2, 1, unroll=2)
        def j_step(j):
            s0 = pl.ds(j * L, L)
            s1 = pl.ds(H + j * L, L)
            for q in range(CQ):
                fb = fsb[q]
                a0 = plsc.bitcast(rt[b, q, pl.ds(j * L, L)], jnp.float32)
                a1 = plsc.bitcast(rt[b, q, pl.ds(H + j * L, L)], jnp.float32)
                bv = plsc.bitcast(rt[b, q, pl.ds(512 + j * L, L)], jnp.bfloat16)
                cv = plsc.bitcast(rt[b, q, pl.ds(768 + j * L, L)], jnp.bfloat16)
                dv = plsc.bitcast(rt[b, q, pl.ds(1024 + j * L, L)], jnp.bfloat16)
                inner = bv + fb * (cv + fb * dv)
                u = plsc.bitcast(inner, jnp.uint32)
                i0 = plsc.bitcast(u << jnp.uint32(16), jnp.float32)
                i1 = plsc.bitcast(u & jnp.uint32(0xFFFF0000), jnp.float32)
                f = fs[q]
                out_v[b, q, s0] = a0 + f * i0
                out_v[b, q, s1] = a1 + f * i1

    fire_gather(0, 0)
    fire_gather(1, 1)

    def pair_step(gg, _):
        for b in range(2):
            g = 2 * gg + b
            wait_gather(g, b)

            @pl.when(gg > 0)
            def _():
                pltpu.make_async_copy(out_v.at[b], out_slice(g - 2), osems[b]).wait()

            compute(g, b)

            @pl.when(g + 2 < NCHUNK)
            def _():
                fire_gather(g + 2, b)

            pltpu.async_copy(out_v.at[b], out_slice(g), osems[b])
        return 0

    lax.fori_loop(0, NCHUNK // 2, pair_step, 0)

    pltpu.make_async_copy(out_v.at[0], out_slice(NCHUNK - 2), osems[0]).wait()
    pltpu.make_async_copy(out_v.at[1], out_slice(NCHUNK - 1), osems[1]).wait()


def _shuffle_bf16(x):
    h = CHANNELS // 2
    lo = lax.bitcast_convert_type(x[:, :h], jnp.uint32) + jnp.uint32(0x8000)
    hi = lax.bitcast_convert_type(x[:, h:], jnp.uint32) + jnp.uint32(0x8000)
    return (lo >> jnp.uint32(16)) | (hi & jnp.uint32(0xFFFF0000))


@jax.jit
def kernel(t, knots, a, b, c, d):
    mesh = plsc.VectorSubcoreMesh(core_axis_name="c", subcore_axis_name="s")
    run = pl.kernel(
        _spline_body,
        out_type=jax.ShapeDtypeStruct((N_QUERY, CHANNELS), jnp.float32),
        mesh=mesh,
        compiler_params=pltpu.CompilerParams(needs_layout_passes=False),
        scratch_types=[
            pltpu.VMEM((QPW,), jnp.float32),
            pltpu.VMEM((QPW,), jnp.int32),
            pltpu.VMEM((QPW,), jnp.float32),
            pltpu.VMEM((N_KNOTS,), jnp.float32),
            pltpu.VMEM((2, CQ, 1280), jnp.uint32),
            pltpu.VMEM((2, CQ, CHANNELS), jnp.float32),
            pltpu.SemaphoreType.DMA,
            pltpu.SemaphoreType.DMA,
            pltpu.SemaphoreType.DMA,
            pltpu.SemaphoreType.DMA,
        ],
    )
    ft = jnp.concatenate(
        [lax.bitcast_convert_type(a, jnp.uint32),
         _shuffle_bf16(b), _shuffle_bf16(c), _shuffle_bf16(d)], axis=1)
    return run(t, knots, ft)


# speedup vs baseline: 1.4525x; 1.0216x over previous
import jax
import jax.numpy as jnp
from jax import lax
from jax.experimental import pallas as pl
from jax.experimental.pallas import tpu as pltpu
from jax.experimental.pallas import tpu_sc as plsc

N_KNOTS = 2048
N_BIN = N_KNOTS - 1
CHANNELS = 512
N_QUERY = 32768

NC = 2
NS = 16
NW = NC * NS
L = 16

QPW = N_QUERY // NW
CQ = L
NBUF = 4
NCHUNK = QPW // CQ
NJ2 = CHANNELS // (2 * L)


def _spline_body(t_hbm, knots_hbm, ft_hbm, out_hbm,
                 t_v, idx_v, frac_v, knots_v,
                 rt, out_v, gsem0, gsem1, gsem2, gsem3, osem0, osem1):
    wid = lax.axis_index("s") * NC + lax.axis_index("c")
    base = wid * QPW
    gsems = (gsem0, gsem1, gsem2, gsem3)
    osems = (osem0, osem1)

    pltpu.sync_copy(t_hbm.at[pl.ds(base, QPW)], t_v)
    pltpu.sync_copy(knots_hbm, knots_v)

    k0 = jnp.full((L,), knots_v[pl.ds(0, L)][0])
    rng = jnp.full((L,), knots_v[pl.ds(N_KNOTS - L, L)][L - 1]) - k0
    nbin = jnp.full((L,), jnp.float32(N_BIN))
    factor = nbin / rng
    step = rng / nbin

    @plsc.parallel_loop(0, QPW // L, 1, unroll=4)
    def idx_step(i):
        tv = t_v[pl.ds(i * L, L)]
        iv = (factor * tv).astype(jnp.int32)
        idx_v[pl.ds(i * L, L)] = iv
        frac_v[pl.ds(i * L, L)] = tv - (k0 + iv.astype(jnp.float32) * step)

    def fire_gather(g, b):
        pltpu.async_copy(ft_hbm.at[idx_v.at[pl.ds(g * CQ, CQ)]], rt.at[b], gsems[b])

    def wait_gather(g, b):
        pltpu.make_async_copy(ft_hbm.at[idx_v.at[pl.ds(g * CQ, CQ)]], rt.at[b], gsems[b]).wait()

    def out_slice(g):
        return out_hbm.at[pl.ds(base + g * CQ, CQ)]

    def compute(g, b, ob):
        fv = frac_v[pl.ds(g * CQ, CQ)]
        fs = [jnp.full((L,), fv[q]) for q in range(CQ)]
        fsb = []
        for q in range(CQ):
            h = (plsc.bitcast(fs[q], jnp.uint32) + jnp.uint32(0x8000)) >> jnp.uint32(16)
            fsb.append(plsc.bitcast((h << jnp.uint32(16)) | h, jnp.bfloat16))

        H = CHANNELS // 2

        @plsc.parallel_loop(0, NJ2, 1, unroll=2)
        def j_step(j):
            s0 = pl.ds(j * L, L)
            s1 = pl.ds(H + j * L, L)
            for q in range(CQ):
                fb = fsb[q]
                a0 = plsc.bitcast(rt[b, q, pl.ds(j * L, L)], jnp.float32)
                a1 = plsc.bitcast(rt[b, q, pl.ds(H + j * L, L)], jnp.float32)
                bv = plsc.bitcast(rt[b, q, pl.ds(512 + j * L, L)], jnp.bfloat16)
                cv = plsc.bitcast(rt[b, q, pl.ds(768 + j * L, L)], jnp.bfloat16)
                dv = plsc.bitcast(rt[b, q, pl.ds(1024 + j * L, L)], jnp.bfloat16)
                inner = bv + fb * (cv + fb * dv)
                u = plsc.bitcast(inner, jnp.uint32)
                i0 = plsc.bitcast(u << jnp.uint32(16), jnp.float32)
                i1 = plsc.bitcast(u & jnp.uint32(0xFFFF0000), jnp.float32)
                f = fs[q]
                out_v[ob, q, s0] = a0 + f * i0
                out_v[ob, q, s1] = a1 + f * i1

    for b in range(NBUF):
        fire_gather(b, b)

    def ring_step(gg, _):
        for b in range(NBUF):
            g = NBUF * gg + b
            ob = b % 2
            wait_gather(g, b)

            @pl.when(gg > 0)
            def _():
                pltpu.make_async_copy(out_v.at[ob], out_slice(g - 2), osems[ob]).wait()

            compute(g, b, ob)

            @pl.when(g + NBUF < NCHUNK)
            def _():
                fire_gather(g + NBUF, b)

            pltpu.async_copy(out_v.at[ob], out_slice(g), osems[ob])
        return 0

    lax.fori_loop(0, NCHUNK // NBUF, ring_step, 0)

    pltpu.make_async_copy(out_v.at[0], out_slice(NCHUNK - 2), osems[0]).wait()
    pltpu.make_async_copy(out_v.at[1], out_slice(NCHUNK - 1), osems[1]).wait()


def _shuffle_bf16(x):
    h = CHANNELS // 2
    lo = lax.bitcast_convert_type(x[:, :h], jnp.uint32) + jnp.uint32(0x8000)
    hi = lax.bitcast_convert_type(x[:, h:], jnp.uint32) + jnp.uint32(0x8000)
    return (lo >> jnp.uint32(16)) | (hi & jnp.uint32(0xFFFF0000))


@jax.jit
def kernel(t, knots, a, b, c, d):
    mesh = plsc.VectorSubcoreMesh(core_axis_name="c", subcore_axis_name="s")
    run = pl.kernel(
        _spline_body,
        out_type=jax.ShapeDtypeStruct((N_QUERY, CHANNELS), jnp.float32),
        mesh=mesh,
        compiler_params=pltpu.CompilerParams(needs_layout_passes=False),
        scratch_types=[
            pltpu.VMEM((QPW,), jnp.float32),
            pltpu.VMEM((QPW,), jnp.int32),
            pltpu.VMEM((QPW,), jnp.float32),
            pltpu.VMEM((N_KNOTS,), jnp.float32),
            pltpu.VMEM((NBUF, CQ, 1280), jnp.uint32),
            pltpu.VMEM((2, CQ, CHANNELS), jnp.float32),
            pltpu.SemaphoreType.DMA,
            pltpu.SemaphoreType.DMA,
            pltpu.SemaphoreType.DMA,
            pltpu.SemaphoreType.DMA,
            pltpu.SemaphoreType.DMA,
            pltpu.SemaphoreType.DMA,
        ],
    )
    ft = jnp.concatenate(
        [lax.bitcast_convert_type(a, jnp.uint32),
         _shuffle_bf16(b), _shuffle_bf16(c), _shuffle_bf16(d)], axis=1)
    return run(t, knots, ft)
